# Initial kernel scaffold; baseline (speedup 1.0000x reference)
import functools

import jax
import jax.numpy as jnp
from jax import lax
from jax.experimental import pallas as pl
from jax.experimental.pallas import tpu as pltpu
from jax.experimental.pallas import tpu_sc as plsc

N_NODES = 10000
NFEAT = 256
NHID = 256
NCLASS = 64
N_EDGES = 160000

NC = 2
NS = 16
CH = 128
CHUNKS = -(-N_EDGES // (NS * CH))
E_PAD = NS * CHUNKS * CH
ROWS_PER_TEC = N_NODES // NS
ZROWS = 125


def _make_sc_agg(d_half: int):
    mesh = plsc.VectorSubcoreMesh(core_axis_name="c", subcore_axis_name="s")
    nk = d_half // 16

    @functools.partial(
        pl.kernel,
        out_type=jax.ShapeDtypeStruct((NC, N_NODES, d_half), jnp.float32),
        mesh=mesh,
        scratch_types=[
            pltpu.VMEM((CHUNKS, CH), jnp.int32),
            pltpu.VMEM((CHUNKS, CH), jnp.int32),
            pltpu.VMEM((CHUNKS, CH), jnp.float32),
            pltpu.VMEM((CH, d_half), jnp.float32),
            pltpu.VMEM((ZROWS, d_half), jnp.float32),
            pltpu.MemorySpace.VMEM_SHARED((N_NODES, d_half), jnp.float32),
            pltpu.SemaphoreType.DMA,
        ],
    )
    def sc_agg(h_hbm, src_hbm, dst_hbm, ew_hbm, out_hbm,
               src_v, dst_v, ew_v, rows_v, zero_v, acc, sem):
        c = lax.axis_index("c")
        s = lax.axis_index("s")

        def zrow(i, _):
            for k in range(nk):
                zero_v[i, pl.ds(k * 16, 16)] = jnp.zeros((16,), jnp.float32)
            return 0
        lax.fori_loop(0, ZROWS, zrow, 0)
        base = s * ROWS_PER_TEC
        for t in range(ROWS_PER_TEC // ZROWS):
            pltpu.sync_copy(zero_v, acc.at[pl.ds(base + t * ZROWS, ZROWS)])

        pltpu.sync_copy(src_hbm.at[s], src_v)
        pltpu.sync_copy(dst_hbm.at[s], dst_v)
        pltpu.sync_copy(ew_hbm.at[s], ew_v)
        plsc.subcore_barrier()

        def chunk(j, _):
            pltpu.async_copy(h_hbm.at[c].at[src_v.at[j]], rows_v, sem).wait()

            def srow(i, _):
                w = ew_v[j, i]
                for k in range(nk):
                    rows_v[i, pl.ds(k * 16, 16)] = (
                        rows_v[i, pl.ds(k * 16, 16)] * w)
                return 0
            lax.fori_loop(0, CH, srow, 0)
            pltpu.sync_copy(rows_v, acc.at[dst_v.at[j]], add=True)
            return 0
        lax.fori_loop(0, CHUNKS, chunk, 0)

        plsc.subcore_barrier()
        pltpu.sync_copy(acc.at[pl.ds(base, ROWS_PER_TEC)],
                        out_hbm.at[c].at[pl.ds(base, ROWS_PER_TEC)])

    return sc_agg


def _mm1(x, w1):
    bm = 1000

    def body(x_ref, w_ref, o_ref):
        o_ref[0] = jnp.dot(x_ref[...], w_ref[...],
                           preferred_element_type=jnp.float32)

    return pl.pallas_call(
        body,
        grid=(N_NODES // bm, NC),
        in_specs=[
            pl.BlockSpec((bm, NFEAT), lambda i, f: (i, 0)),
            pl.BlockSpec((NFEAT, NHID // NC), lambda i, f: (0, f)),
        ],
        out_specs=pl.BlockSpec((1, bm, NHID // NC), lambda i, f: (f, i, 0)),
        out_shape=jax.ShapeDtypeStruct((NC, N_NODES, NHID // NC), jnp.float32),
    )(x, w1)


def _mm2(a, w2):
    bm = 1000
    dh = NHID // NC
    ch = NCLASS // NC

    def body(a_ref, w_ref, o_ref):
        ar = jax.nn.relu(a_ref[...])
        full = (jnp.dot(ar[0], w_ref[:dh, :], preferred_element_type=jnp.float32)
                + jnp.dot(ar[1], w_ref[dh:, :], preferred_element_type=jnp.float32))
        o_ref[0] = full[:, :ch]
        o_ref[1] = full[:, ch:]

    return pl.pallas_call(
        body,
        grid=(N_NODES // bm,),
        in_specs=[
            pl.BlockSpec((NC, bm, dh), lambda i: (0, i, 0)),
            pl.BlockSpec((NHID, NCLASS), lambda i: (0, 0)),
        ],
        out_specs=pl.BlockSpec((NC, bm, ch), lambda i: (0, i, 0)),
        out_shape=jax.ShapeDtypeStruct((NC, N_NODES, ch), jnp.float32),
    )(a, w2)


def _prep_edges(edge_index, edge_weight):
    src = edge_index[0].astype(jnp.int32)
    dst = edge_index[1].astype(jnp.int32)
    pad = E_PAD - N_EDGES
    fill = (jnp.arange(pad, dtype=jnp.int32) * 37) % N_NODES
    src = jnp.concatenate([src, fill]).reshape(NS, CHUNKS, CH)
    dst = jnp.concatenate([dst, fill]).reshape(NS, CHUNKS, CH)
    ew = jnp.concatenate(
        [edge_weight, jnp.zeros((pad,), jnp.float32)]).reshape(NS, CHUNKS, CH)
    return src, dst, ew


_sc_agg_128 = _make_sc_agg(NHID // NC)
_sc_agg_32 = _make_sc_agg(NCLASS // NC)


def kernel(x, edge_index1, edge_index2, edge_weight1, edge_weight2, W1, W2):
    src1, dst1, ew1 = _prep_edges(edge_index1, edge_weight1)
    src2, dst2, ew2 = _prep_edges(edge_index2, edge_weight2)

    h = _mm1(x, W1)
    a1 = _sc_agg_128(h, src1, dst1, ew1)
    h2 = _mm2(a1, W2)
    a2 = _sc_agg_32(h2, src2, dst2, ew2)
    return jnp.moveaxis(a2, 0, 1).reshape(N_NODES, NCLASS)


# speedup vs baseline: 4.7171x; 4.7171x over previous
import functools

import jax
import jax.numpy as jnp
from jax import lax
from jax.experimental import pallas as pl
from jax.experimental.pallas import tpu as pltpu
from jax.experimental.pallas import tpu_sc as plsc

N_NODES = 10000
NFEAT = 256
NHID = 256
NCLASS = 64
N_EDGES = 160000

NC = 2
NS = 16
CH = 128
CHUNKS = -(-N_EDGES // (NS * CH))
E_PAD = NS * CHUNKS * CH
N_PAD = 10112
ROWS_PER_TEC = N_PAD // NS
ZROWS = 8


def _make_sc_agg(d_half: int, tc_tiling: bool = True):
    mesh = plsc.VectorSubcoreMesh(core_axis_name="c", subcore_axis_name="s")
    nk = d_half // 16

    @functools.partial(
        pl.kernel,
        out_type=jax.ShapeDtypeStruct((NC, N_PAD, d_half), jnp.float32),
        mesh=mesh,
        scratch_types=[
            pltpu.VMEM((CHUNKS, CH), jnp.int32),
            pltpu.VMEM((CHUNKS, CH), jnp.int32),
            pltpu.VMEM((CHUNKS, CH), jnp.float32),
            pltpu.VMEM((CH, d_half), jnp.float32),
            pltpu.VMEM((ZROWS, d_half), jnp.float32),
            pltpu.MemorySpace.VMEM_SHARED((N_PAD, d_half), jnp.float32),
            pltpu.SemaphoreType.DMA,
        ],
        compiler_params=pltpu.CompilerParams(use_tc_tiling_on_sc=tc_tiling),
    )
    def sc_agg(h_hbm, src_hbm, dst_hbm, ew_hbm, out_hbm,
               src_v, dst_v, ew_v, rows_v, zero_v, acc, sem):
        c = lax.axis_index("c")
        s = lax.axis_index("s")

        for i in range(ZROWS):
            for k in range(nk):
                zero_v[i, pl.ds(k * 16, 16)] = jnp.zeros((16,), jnp.float32)
        base = s * ROWS_PER_TEC

        def zblk(t, _):
            pltpu.sync_copy(zero_v, acc.at[pl.ds(base + t * ZROWS, ZROWS)])
            return 0
        lax.fori_loop(0, ROWS_PER_TEC // ZROWS, zblk, 0)

        pltpu.sync_copy(src_hbm.at[s], src_v)
        pltpu.sync_copy(dst_hbm.at[s], dst_v)
        pltpu.sync_copy(ew_hbm.at[s], ew_v)
        plsc.subcore_barrier()

        def chunk(j, _):
            pltpu.async_copy(h_hbm.at[c].at[src_v.at[j]], rows_v, sem).wait()

            def sgroup(g, _):
                wv = ew_v[j, pl.ds(g * 16, 16)]
                r0 = g * 16
                for i in range(16):
                    w = wv[i]
                    for k in range(nk):
                        rows_v[r0 + i, pl.ds(k * 16, 16)] = (
                            rows_v[r0 + i, pl.ds(k * 16, 16)] * w)
                return 0
            lax.fori_loop(0, CH // 16, sgroup, 0)
            pltpu.sync_copy(rows_v, acc.at[dst_v.at[j]], add=True)
            return 0
        lax.fori_loop(0, CHUNKS, chunk, 0)

        plsc.subcore_barrier()
        pltpu.sync_copy(acc.at[pl.ds(base, ROWS_PER_TEC)],
                        out_hbm.at[c].at[pl.ds(base, ROWS_PER_TEC)])

    return sc_agg


def _mm1(x, w1):
    bm = 1000

    def body(x_ref, w_ref, o_ref):
        o_ref[0] = jnp.dot(x_ref[...], w_ref[...],
                           preferred_element_type=jnp.float32)

    return pl.pallas_call(
        body,
        grid=(N_NODES // bm, NC),
        in_specs=[
            pl.BlockSpec((bm, NFEAT), lambda i, f: (i, 0)),
            pl.BlockSpec((NFEAT, NHID // NC), lambda i, f: (0, f)),
        ],
        out_specs=pl.BlockSpec((1, bm, NHID // NC), lambda i, f: (f, i, 0)),
        out_shape=jax.ShapeDtypeStruct((NC, N_NODES, NHID // NC), jnp.float32),
    )(x, w1)


def _mm2(a, w2):
    bm = 632
    dh = NHID // NC
    ch = NCLASS // NC

    def body(a_ref, w_ref, o_ref):
        ar = jax.nn.relu(a_ref[...])
        full = (jnp.dot(ar[0], w_ref[:dh, :], preferred_element_type=jnp.float32)
                + jnp.dot(ar[1], w_ref[dh:, :], preferred_element_type=jnp.float32))
        o_ref[0] = full[:, :ch]
        o_ref[1] = full[:, ch:]

    return pl.pallas_call(
        body,
        grid=(N_PAD // bm,),
        in_specs=[
            pl.BlockSpec((NC, bm, dh), lambda i: (0, i, 0)),
            pl.BlockSpec((NHID, NCLASS), lambda i: (0, 0)),
        ],
        out_specs=pl.BlockSpec((NC, bm, ch), lambda i: (0, i, 0)),
        out_shape=jax.ShapeDtypeStruct((NC, N_PAD, ch), jnp.float32),
    )(a, w2)


def _prep_edges(edge_index, edge_weight):
    src = edge_index[0].astype(jnp.int32)
    dst = edge_index[1].astype(jnp.int32)
    pad = E_PAD - N_EDGES
    fill = (jnp.arange(pad, dtype=jnp.int32) * 37) % N_NODES
    src = jnp.concatenate([src, fill]).reshape(NS, CHUNKS, CH)
    dst = jnp.concatenate([dst, fill]).reshape(NS, CHUNKS, CH)
    ew = jnp.concatenate(
        [edge_weight, jnp.zeros((pad,), jnp.float32)]).reshape(NS, CHUNKS, CH)
    return src, dst, ew


_sc_agg_128 = _make_sc_agg(NHID // NC)
_sc_agg_32 = _make_sc_agg(NCLASS // NC, tc_tiling=False)


def kernel(x, edge_index1, edge_index2, edge_weight1, edge_weight2, W1, W2):
    src1, dst1, ew1 = _prep_edges(edge_index1, edge_weight1)
    src2, dst2, ew2 = _prep_edges(edge_index2, edge_weight2)

    h = _mm1(x, W1)
    a1 = _sc_agg_128(h, src1, dst1, ew1)
    h2 = _mm2(a1, W2)
    a2 = _sc_agg_32(h2, src2, dst2, ew2)
    return jnp.moveaxis(a2[:, :N_NODES], 0, 1).reshape(N_NODES, NCLASS)


# speedup vs baseline: 7.6721x; 1.6264x over previous
import functools

import jax
import jax.numpy as jnp
from jax import lax
from jax.experimental import pallas as pl
from jax.experimental.pallas import tpu as pltpu
from jax.experimental.pallas import tpu_sc as plsc

N_NODES = 10000
NFEAT = 256
NHID = 256
NCLASS = 64
N_EDGES = 160000

NC = 2
NS = 16
CH = 80
G = 8
NBUF = 4
CHUNKS = 128
NSLAB = CHUNKS // G
E_PAD = NS * CHUNKS * CH
N_PAD = 10112
ROWS_PER_TEC = N_PAD // NS
ZROWS = 8


def _make_sc_agg(d_half: int, tc_tiling: bool = True):
    mesh = plsc.VectorSubcoreMesh(core_axis_name="c", subcore_axis_name="s")
    nk = d_half // 16

    @functools.partial(
        pl.kernel,
        out_type=jax.ShapeDtypeStruct((NC, N_PAD, d_half), jnp.float32),
        mesh=mesh,
        scratch_types=[
            pltpu.VMEM((2, G, CH), jnp.int32),
            pltpu.VMEM((2, G, CH), jnp.int32),
            pltpu.VMEM((2, G, CH), jnp.float32),
            pltpu.VMEM((NBUF, CH, d_half), jnp.float32),
            pltpu.VMEM((ZROWS, d_half), jnp.float32),
            pltpu.MemorySpace.VMEM_SHARED((N_PAD, d_half), jnp.float32),
            pltpu.SemaphoreType.DMA((NBUF,)),
            pltpu.SemaphoreType.DMA((NBUF,)),
            pltpu.SemaphoreType.DMA,
            pltpu.SemaphoreType.DMA,
        ],
        compiler_params=pltpu.CompilerParams(use_tc_tiling_on_sc=tc_tiling),
    )
    def sc_agg(h_hbm, src_hbm, dst_hbm, ew_hbm, out_hbm,
               esrc, edst, eww, rows, zero_v, acc, gsem, ssem, esem, zsem):
        c = lax.axis_index("c")
        s = lax.axis_index("s")

        def wait_gather(b):
            pltpu.make_async_copy(
                h_hbm.at[c].at[pl.ds(0, CH)], rows.at[b], gsem.at[b]).wait()

        def wait_scatter(b):
            pltpu.make_async_copy(
                h_hbm.at[c].at[pl.ds(0, CH)], rows.at[b], ssem.at[b]).wait()

        def start_gather(idx_row, b):
            pltpu.make_async_copy(
                h_hbm.at[c].at[idx_row], rows.at[b], gsem.at[b]).start()

        for i in range(ZROWS):
            for k in range(nk):
                zero_v[i, pl.ds(k * 16, 16)] = jnp.zeros((16,), jnp.float32)
        base = s * ROWS_PER_TEC
        nz = ROWS_PER_TEC // ZROWS

        def zstart(t, _):
            pltpu.make_async_copy(
                zero_v, acc.at[pl.ds(base + t * ZROWS, ZROWS)], zsem).start()
            return 0
        lax.fori_loop(0, nz, zstart, 0)

        pltpu.sync_copy(src_hbm.at[s, 0], esrc.at[0])
        pltpu.sync_copy(dst_hbm.at[s, 0], edst.at[0])
        pltpu.sync_copy(ew_hbm.at[s, 0], eww.at[0])
        start_gather(esrc.at[0, 0], 0)
        start_gather(esrc.at[0, 1], 1)

        pltpu.make_async_copy(
            h_hbm.at[c].at[pl.ds(0, ROWS_PER_TEC)],
            acc.at[pl.ds(base, ROWS_PER_TEC)], zsem).wait()
        plsc.subcore_barrier()

        def slab(t, _):
            a = t % 2
            na = (t + 1) % 2

            for q in range(G):
                if q == 2:
                    @pl.when(t + 1 < NSLAB)
                    def _():
                        pltpu.make_async_copy(
                            src_hbm.at[s, t + 1], esrc.at[na], esem).start()
                        pltpu.make_async_copy(
                            dst_hbm.at[s, t + 1], edst.at[na], esem).start()
                        pltpu.make_async_copy(
                            ew_hbm.at[s, t + 1], eww.at[na], esem).start()
                b = q % NBUF
                bn = (q + 2) % NBUF
                if q == 6:
                    @pl.when(t + 1 < NSLAB)
                    def _():
                        pltpu.make_async_copy(
                            src_hbm.at[s, 0], esrc.at[0], esem).wait()
                        pltpu.make_async_copy(
                            dst_hbm.at[s, 0], edst.at[0], esem).wait()
                        pltpu.make_async_copy(
                            ew_hbm.at[s, 0], eww.at[0], esem).wait()

                if q < 6:
                    if q < 2:
                        @pl.when(t > 0)
                        def _():
                            wait_scatter(bn)
                    else:
                        wait_scatter(bn)
                    start_gather(esrc.at[a, q + 2], bn)
                else:
                    @pl.when(t + 1 < NSLAB)
                    def _():
                        wait_scatter(bn)
                        start_gather(esrc.at[na, q - 6], bn)

                wait_gather(b)

                def sgroup(g, _, b=b, a=a, q=q):
                    wv = eww[a, q, pl.ds(g * 16, 16)]
                    for i in range(16):
                        w = wv[i]
                        r = g * 16 + i
                        for k in range(nk):
                            rows[b, r, pl.ds(k * 16, 16)] = (
                                rows[b, r, pl.ds(k * 16, 16)] * w)
                    return 0
                lax.fori_loop(0, CH // 16, sgroup, 0)

                pltpu.make_async_copy(
                    rows.at[b], acc.at[edst.at[a, q]],
                    ssem.at[b]).start(add=True)
            return 0
        lax.fori_loop(0, NSLAB, slab, 0)

        for b in range(NBUF):
            wait_scatter(b)
        plsc.subcore_barrier()
        pltpu.sync_copy(acc.at[pl.ds(base, ROWS_PER_TEC)],
                        out_hbm.at[c].at[pl.ds(base, ROWS_PER_TEC)])

    return sc_agg


def _mm1(x, w1):
    bm = 1000

    def body(x_ref, w_ref, o_ref):
        o_ref[0] = jnp.dot(x_ref[...], w_ref[...],
                           preferred_element_type=jnp.float32)

    return pl.pallas_call(
        body,
        grid=(N_NODES // bm, NC),
        in_specs=[
            pl.BlockSpec((bm, NFEAT), lambda i, f: (i, 0)),
            pl.BlockSpec((NFEAT, NHID // NC), lambda i, f: (0, f)),
        ],
        out_specs=pl.BlockSpec((1, bm, NHID // NC), lambda i, f: (f, i, 0)),
        out_shape=jax.ShapeDtypeStruct((NC, N_NODES, NHID // NC), jnp.float32),
    )(x, w1)


def _mm2(a, w2):
    bm = 632
    dh = NHID // NC
    ch = NCLASS // NC

    def body(a_ref, w_ref, o_ref):
        ar = jax.nn.relu(a_ref[...])
        full = (jnp.dot(ar[0], w_ref[:dh, :], preferred_element_type=jnp.float32)
                + jnp.dot(ar[1], w_ref[dh:, :], preferred_element_type=jnp.float32))
        o_ref[0] = full[:, :ch]
        o_ref[1] = full[:, ch:]

    return pl.pallas_call(
        body,
        grid=(N_PAD // bm,),
        in_specs=[
            pl.BlockSpec((NC, bm, dh), lambda i: (0, i, 0)),
            pl.BlockSpec((NHID, NCLASS), lambda i: (0, 0)),
        ],
        out_specs=pl.BlockSpec((NC, bm, ch), lambda i: (0, i, 0)),
        out_shape=jax.ShapeDtypeStruct((NC, N_PAD, ch), jnp.float32),
    )(a, w2)


def _prep_edges(edge_index, edge_weight):
    src = edge_index[0].astype(jnp.int32)
    dst = edge_index[1].astype(jnp.int32)
    pad = E_PAD - N_EDGES
    fill = (jnp.arange(pad, dtype=jnp.int32) * 37) % N_NODES
    src = jnp.concatenate([src, fill]).reshape(NS, NSLAB, G, CH)
    dst = jnp.concatenate([dst, fill]).reshape(NS, NSLAB, G, CH)
    ew = jnp.concatenate(
        [edge_weight, jnp.zeros((pad,), jnp.float32)]).reshape(NS, NSLAB, G, CH)
    return src, dst, ew


_sc_agg_128 = _make_sc_agg(NHID // NC)
_sc_agg_32 = _make_sc_agg(NCLASS // NC, tc_tiling=False)


def kernel(x, edge_index1, edge_index2, edge_weight1, edge_weight2, W1, W2):
    src1, dst1, ew1 = _prep_edges(edge_index1, edge_weight1)
    src2, dst2, ew2 = _prep_edges(edge_index2, edge_weight2)

    h = _mm1(x, W1)
    a1 = _sc_agg_128(h, src1, dst1, ew1)
    h2 = _mm2(a1, W2)
    a2 = _sc_agg_32(h2, src2, dst2, ew2)
    return jnp.moveaxis(a2[:, :N_NODES], 0, 1).reshape(N_NODES, NCLASS)
